# baseline (speedup 1.0000x reference)
import functools

import jax
import jax.numpy as jnp
from jax import lax
from jax.experimental import pallas as pl
from jax.experimental.pallas import tpu as pltpu

D_MODEL = 2048
D_SPACE = 64
N_PER_TABLE = 8192
TOKENS = 4096
TB = 256
CW = 512
COL = 16
NFOLD = 4


def _cmpx(av, ai, bv, bi, desc):
    ge = av >= bv
    hv = jnp.maximum(av, bv)
    lv = jnp.minimum(av, bv)
    hi = jnp.where(ge, ai, bi)
    li = jnp.where(ge, bi, ai)
    if desc:
        return (hv, hi), (lv, li)
    return (lv, li), (hv, hi)


def _bitonic_stages(n):
    out = []
    k = 2
    while k <= n:
        j = k // 2
        while j >= 1:
            out.append((k, j))
            j //= 2
        k *= 2
    return out


def _sort_columns_desc(v, i, n):
    for k, j in _bitonic_stages(n):
        for r in range(0, n, 2 * j):
            for c in range(r, r + j):
                desc = (c & k) == 0
                (xv, xi), (yv, yi) = _cmpx(v[c], i[c], v[c + j], i[c + j], desc)
                v[c], i[c] = xv, xi
                v[c + j], i[c + j] = yv, yi
    return v, i


def _bitonic_merge_desc(mv, mi, n):
    j = n // 2
    while j >= 1:
        for r in range(0, n, 2 * j):
            for c in range(r, r + j):
                (xv, xi), (yv, yi) = _cmpx(mv[c], mi[c], mv[c + j], mi[c + j], True)
                mv[c], mi[c] = xv, xi
                mv[c + j], mi[c + j] = yv, yi
        j //= 2
    return mv, mi


def _merge_level(v, i, n, grow, ngroups):
    w = v[0].shape[1]
    g_ = w // ngroups
    h = g_ // 2

    def halves(x, lo):
        parts = [x[:, g * g_ + (0 if lo else h): g * g_ + (h if lo else g_)]
                 for g in range(ngroups)]
        return jnp.concatenate(parts, axis=1) if len(parts) > 1 else parts[0]

    av = [halves(x, True) for x in v]
    ai = [halves(x, True) for x in i]
    bv = [halves(x, False) for x in v]
    bi = [halves(x, False) for x in i]
    mv, mi, lv, li = [], [], [], []
    for c in range(n):
        ge = av[c] >= bv[n - 1 - c]
        mv.append(jnp.maximum(av[c], bv[n - 1 - c]))
        mi.append(jnp.where(ge, ai[c], bi[n - 1 - c]))
        if grow:
            lv.append(jnp.minimum(av[c], bv[n - 1 - c]))
            li.append(jnp.where(ge, bi[n - 1 - c], ai[c]))
    mv, mi = _bitonic_merge_desc(mv, mi, n)
    if grow:
        lv, li = _bitonic_merge_desc(lv, li, n)
        mv += lv
        mi += li
    return mv, mi


def _fold(arrs):
    r = arrs[0].shape[0]
    return [jnp.concatenate([x[:r // 2], x[r // 2:]], axis=1) for x in arrs]


def _unfold(arrs, times):
    for _ in range(times):
        w = arrs[0].shape[1]
        arrs = [jnp.concatenate([x[:, :w // 2], x[:, w // 2:]], axis=0)
                for x in arrs]
    return arrs


def _topk(logits, kk):
    v = [logits[:, c * CW:(c + 1) * CW] for c in range(COL)]
    i = [lax.broadcasted_iota(jnp.int32, (TB, CW), 1) + (c * CW)
         for c in range(COL)]
    v, i = _sort_columns_desc(v, i, COL)
    clen, ngroups, folds = COL, 1, 0
    while v[0].shape[1] // ngroups > 1:
        g_ = v[0].shape[1] // ngroups
        grow = clen < kk
        if (not grow and g_ <= 64 and folds < NFOLD
                and v[0].shape[0] > 8):
            v, i = _fold(v), _fold(i)
            ngroups *= 2
            folds += 1
        v, i = _merge_level(v, i, clen, grow, ngroups)
        if grow:
            clen *= 2
    v, i = _unfold(v, folds), _unfold(i, folds)
    return jnp.concatenate(v, axis=1), jnp.concatenate(i, axis=1)


def _router_body(kk, x_ref, w_ref, b_ref, e_ref, wk_ref, idx_ref, h_ref):
    t = pl.program_id(1)

    @pl.when(t == 0)
    def _():
        h_ref[...] = jnp.dot(x_ref[...], w_ref[...],
                             preferred_element_type=jnp.float32) + b_ref[...]

    h = h_ref[...]
    e = e_ref[0]
    nrm = jnp.sqrt(jnp.sum(e * e, axis=-1, keepdims=True))
    e = e / jnp.maximum(nrm, 1e-12)
    logits = lax.dot_general(h, e, (((1,), (1,)), ((), ())),
                             preferred_element_type=jnp.float32)
    vals, idx = _topk(logits, kk)
    ex = jnp.exp(vals - vals[:, :1])
    wk_ref[0] = ex / jnp.sum(ex, axis=-1, keepdims=True)
    idx_ref[0] = idx


def _route_tables(x2d, W_proj, b2d, tables, kk):
    nt = tables.shape[0]
    return pl.pallas_call(
        functools.partial(_router_body, kk),
        grid=(TOKENS // TB, nt),
        in_specs=[
            pl.BlockSpec((TB, D_MODEL), lambda b, t: (b, 0)),
            pl.BlockSpec((D_MODEL, D_SPACE), lambda b, t: (0, 0)),
            pl.BlockSpec((1, D_SPACE), lambda b, t: (0, 0)),
            pl.BlockSpec((1, N_PER_TABLE, D_SPACE), lambda b, t: (t, 0, 0)),
        ],
        out_specs=[
            pl.BlockSpec((1, TB, kk), lambda b, t: (t, b, 0)),
            pl.BlockSpec((1, TB, kk), lambda b, t: (t, b, 0)),
        ],
        out_shape=[
            jax.ShapeDtypeStruct((nt, TOKENS, kk), jnp.float32),
            jax.ShapeDtypeStruct((nt, TOKENS, kk), jnp.int32),
        ],
        scratch_shapes=[pltpu.VMEM((TB, D_SPACE), jnp.float32)],
    )(x2d, W_proj, b2d, tables)


def kernel(x, W_proj, b_proj, neuron_emb, neuron_emb_feature_v, neuron_emb_relational_k):
    B, S, _ = x.shape
    x2d = x.reshape(B * S, D_MODEL)
    b2d = b_proj.reshape(1, D_SPACE)
    tables64 = jnp.stack([
        neuron_emb[:N_PER_TABLE],
        neuron_emb_feature_v,
        neuron_emb[N_PER_TABLE:2 * N_PER_TABLE],
        neuron_emb_relational_k,
    ])
    tables32 = jnp.stack([
        neuron_emb[2 * N_PER_TABLE:3 * N_PER_TABLE],
        neuron_emb[3 * N_PER_TABLE:],
    ])
    w64, i64 = _route_tables(x2d, W_proj, b2d, tables64, 64)
    w32, i32 = _route_tables(x2d, W_proj, b2d, tables32, 32)
    weights = jnp.concatenate(
        [w64[0], w64[1], w64[2], w64[3], w32[0], w32[1]], axis=-1)
    indices = jnp.concatenate(
        [i64[0], i64[1], i64[2], i64[3], i32[0], i32[1]], axis=-1)
    return (weights.reshape(B, S, -1), indices.reshape(B, S, -1))


# speedup vs baseline: 1.2477x; 1.2477x over previous
import functools

import jax
import jax.numpy as jnp
from jax import lax
from jax.experimental import pallas as pl
from jax.experimental.pallas import tpu as pltpu

D_MODEL = 2048
D_SPACE = 64
N_PER_TABLE = 8192
TOKENS = 4096
TB = 128
CW = 512
COL = 16
NFOLD = 5


def _cmpx(av, ai, bv, bi, desc):
    ge = av >= bv
    hv = jnp.maximum(av, bv)
    lv = jnp.minimum(av, bv)
    hi = jnp.where(ge, ai, bi)
    li = jnp.where(ge, bi, ai)
    if desc:
        return (hv, hi), (lv, li)
    return (lv, li), (hv, hi)


def _bitonic_stages(n):
    out = []
    k = 2
    while k <= n:
        j = k // 2
        while j >= 1:
            out.append((k, j))
            j //= 2
        k *= 2
    return out


def _sort_columns_desc(v, i, n):
    for k, j in _bitonic_stages(n):
        for r in range(0, n, 2 * j):
            for c in range(r, r + j):
                desc = (c & k) == 0
                (xv, xi), (yv, yi) = _cmpx(v[c], i[c], v[c + j], i[c + j], desc)
                v[c], i[c] = xv, xi
                v[c + j], i[c + j] = yv, yi
    return v, i


def _bitonic_merge_desc(mv, mi, n):
    j = n // 2
    while j >= 1:
        for r in range(0, n, 2 * j):
            for c in range(r, r + j):
                (xv, xi), (yv, yi) = _cmpx(mv[c], mi[c], mv[c + j], mi[c + j], True)
                mv[c], mi[c] = xv, xi
                mv[c + j], mi[c + j] = yv, yi
        j //= 2
    return mv, mi


def _merge_level(v, i, n, grow, ngroups):
    w = v[0].shape[1]
    g_ = w // ngroups
    h = g_ // 2

    def halves(x, lo):
        parts = [x[:, g * g_ + (0 if lo else h): g * g_ + (h if lo else g_)]
                 for g in range(ngroups)]
        return jnp.concatenate(parts, axis=1) if len(parts) > 1 else parts[0]

    av = [halves(x, True) for x in v]
    ai = [halves(x, True) for x in i]
    bv = [halves(x, False) for x in v]
    bi = [halves(x, False) for x in i]
    mv, mi, lv, li = [], [], [], []
    for c in range(n):
        ge = av[c] >= bv[n - 1 - c]
        mv.append(jnp.maximum(av[c], bv[n - 1 - c]))
        mi.append(jnp.where(ge, ai[c], bi[n - 1 - c]))
        if grow:
            lv.append(jnp.minimum(av[c], bv[n - 1 - c]))
            li.append(jnp.where(ge, bi[n - 1 - c], ai[c]))
    mv, mi = _bitonic_merge_desc(mv, mi, n)
    if grow:
        lv, li = _bitonic_merge_desc(lv, li, n)
        mv += lv
        mi += li
    return mv, mi


def _fold(arrs):
    r = arrs[0].shape[0]
    return [jnp.concatenate([x[:r // 2], x[r // 2:]], axis=1) for x in arrs]


def _unfold(arrs, times):
    for _ in range(times):
        w = arrs[0].shape[1]
        arrs = [jnp.concatenate([x[:, :w // 2], x[:, w // 2:]], axis=0)
                for x in arrs]
    return arrs


def _topk(logits, kk):
    v = [logits[:, c * CW:(c + 1) * CW] for c in range(COL)]
    i = [lax.broadcasted_iota(jnp.int32, (TB, CW), 1) + (c * CW)
         for c in range(COL)]
    v, i = _sort_columns_desc(v, i, COL)
    clen, ngroups, folds = COL, 1, 0
    while v[0].shape[1] // ngroups > 1:
        g_ = v[0].shape[1] // ngroups
        grow = clen < kk
        if (not grow and g_ <= 128 and folds < NFOLD
                and v[0].shape[0] > 8):
            v, i = _fold(v), _fold(i)
            ngroups *= 2
            folds += 1
        v, i = _merge_level(v, i, clen, grow, ngroups)
        if grow:
            clen *= 2
    v, i = _unfold(v, folds), _unfold(i, folds)
    return jnp.concatenate(v, axis=1), jnp.concatenate(i, axis=1)


def _router_body(kk, x_ref, w_ref, b_ref, e_ref, wk_ref, idx_ref, h_ref):
    t = pl.program_id(1)

    @pl.when(t == 0)
    def _():
        h_ref[...] = jnp.dot(x_ref[...], w_ref[...],
                             preferred_element_type=jnp.float32) + b_ref[...]

    h = h_ref[...]
    e = e_ref[0]
    nrm = jnp.sqrt(jnp.sum(e * e, axis=-1, keepdims=True))
    e = e / jnp.maximum(nrm, 1e-12)
    logits = lax.dot_general(h, e, (((1,), (1,)), ((), ())),
                             preferred_element_type=jnp.float32)
    vals, idx = _topk(logits, kk)
    ex = jnp.exp(vals - vals[:, :1])
    wk_ref[0] = ex / jnp.sum(ex, axis=-1, keepdims=True)
    idx_ref[0] = idx


def _route_tables(x2d, W_proj, b2d, tables, kk):
    nt = tables.shape[0]
    return pl.pallas_call(
        functools.partial(_router_body, kk),
        grid=(TOKENS // TB, nt),
        in_specs=[
            pl.BlockSpec((TB, D_MODEL), lambda b, t: (b, 0)),
            pl.BlockSpec((D_MODEL, D_SPACE), lambda b, t: (0, 0)),
            pl.BlockSpec((1, D_SPACE), lambda b, t: (0, 0)),
            pl.BlockSpec((1, N_PER_TABLE, D_SPACE), lambda b, t: (t, 0, 0)),
        ],
        out_specs=[
            pl.BlockSpec((1, TB, kk), lambda b, t: (t, b, 0)),
            pl.BlockSpec((1, TB, kk), lambda b, t: (t, b, 0)),
        ],
        out_shape=[
            jax.ShapeDtypeStruct((nt, TOKENS, kk), jnp.float32),
            jax.ShapeDtypeStruct((nt, TOKENS, kk), jnp.int32),
        ],
        scratch_shapes=[pltpu.VMEM((TB, D_SPACE), jnp.float32)],
    )(x2d, W_proj, b2d, tables)


def kernel(x, W_proj, b_proj, neuron_emb, neuron_emb_feature_v, neuron_emb_relational_k):
    B, S, _ = x.shape
    x2d = x.reshape(B * S, D_MODEL)
    b2d = b_proj.reshape(1, D_SPACE)
    tables64 = jnp.stack([
        neuron_emb[:N_PER_TABLE],
        neuron_emb_feature_v,
        neuron_emb[N_PER_TABLE:2 * N_PER_TABLE],
        neuron_emb_relational_k,
    ])
    tables32 = jnp.stack([
        neuron_emb[2 * N_PER_TABLE:3 * N_PER_TABLE],
        neuron_emb[3 * N_PER_TABLE:],
    ])
    w64, i64 = _route_tables(x2d, W_proj, b2d, tables64, 64)
    w32, i32 = _route_tables(x2d, W_proj, b2d, tables32, 32)
    weights = jnp.concatenate(
        [w64[0], w64[1], w64[2], w64[3], w32[0], w32[1]], axis=-1)
    indices = jnp.concatenate(
        [i64[0], i64[1], i64[2], i64[3], i32[0], i32[1]], axis=-1)
    return (weights.reshape(B, S, -1), indices.reshape(B, S, -1))
